# Initial kernel scaffold; baseline (speedup 1.0000x reference)
import jax
import jax.numpy as jnp
from jax.experimental import pallas as pl


def kernel(x, table):
    raise NotImplementedError("write your pallas kernel here")


# speedup vs baseline: 1.5607x; 1.5607x over previous
import functools

import jax
import jax.numpy as jnp
from jax import lax
from jax.experimental import pallas as pl
from jax.experimental.pallas import tpu as pltpu
from jax.experimental.pallas import tpu_sc as plsc

_ROWS = 1000000
_D = 32
_B = 16384
_F = 26
_TOTAL = _B * _F
_NC = 2
_NS = 16
_NW = _NC * _NS
_B_PER_W = _TOTAL // _NW
_CHUNK = 1664
_NCHUNK = _B_PER_W // _CHUNK

_mesh = plsc.VectorSubcoreMesh(core_axis_name="c", subcore_axis_name="s")


@functools.partial(
    pl.kernel,
    out_type=jax.ShapeDtypeStruct((_TOTAL, _D), jnp.float32),
    mesh=_mesh,
    scratch_types=[
        pltpu.VMEM((_CHUNK,), jnp.int32),
        pltpu.VMEM((_CHUNK, _D), jnp.float32),
        pltpu.SemaphoreType.DMA,
    ],
    compiler_params=pltpu.CompilerParams(use_tc_tiling_on_sc=False),
)
def _embed_gather(x_hbm, table_hbm, out_hbm, idx_v, rows_v, sem):
    wid = lax.axis_index("s") * _NC + lax.axis_index("c")
    base = wid * _B_PER_W
    for i in range(_NCHUNK):
        off = base + i * _CHUNK
        pltpu.sync_copy(x_hbm.at[pl.ds(off, _CHUNK)], idx_v)
        pltpu.async_copy(table_hbm.at[idx_v], rows_v, sem).wait()
        pltpu.sync_copy(rows_v, out_hbm.at[pl.ds(off, _CHUNK)])


def kernel(x, table):
    flat = x.reshape(_TOTAL).astype(jnp.int32)
    out = _embed_gather(flat, table)
    return out.reshape(_B, _F, _D)


# speedup vs baseline: 1.5758x; 1.0097x over previous
import functools

import jax
import jax.numpy as jnp
from jax import lax
from jax.experimental import pallas as pl
from jax.experimental.pallas import tpu as pltpu
from jax.experimental.pallas import tpu_sc as plsc

_ROWS = 1000000
_D = 32
_B = 16384
_F = 26
_TOTAL = _B * _F
_NC = 2
_NS = 16
_NW = _NC * _NS
_B_PER_W = _TOTAL // _NW
_CHUNK = 832
_NCHUNK = _B_PER_W // _CHUNK
_NBUF = 4

_mesh = plsc.VectorSubcoreMesh(core_axis_name="c", subcore_axis_name="s")


@functools.partial(
    pl.kernel,
    out_type=jax.ShapeDtypeStruct((_TOTAL, _D), jnp.float32),
    mesh=_mesh,
    scratch_types=[
        pltpu.VMEM((_NCHUNK, _CHUNK), jnp.int32),
        [pltpu.VMEM((_CHUNK, _D), jnp.float32) for _ in range(_NBUF)],
        [pltpu.SemaphoreType.DMA for _ in range(_NBUF)],
        [pltpu.SemaphoreType.DMA for _ in range(_NBUF)],
    ],
    compiler_params=pltpu.CompilerParams(use_tc_tiling_on_sc=False),
)
def _embed_gather(x_hbm, table_hbm, out_hbm, idx_v, bufs, gsems, ssems):
    wid = lax.axis_index("s") * _NC + lax.axis_index("c")
    base = wid * _B_PER_W
    pltpu.sync_copy(x_hbm.at[pl.ds(wid * _NCHUNK, _NCHUNK)], idx_v)

    gd = [None] * _NCHUNK
    sd = [None] * _NCHUNK
    for i in range(min(_NBUF, _NCHUNK)):
        gd[i] = pltpu.async_copy(table_hbm.at[idx_v.at[i]], bufs[i], gsems[i])
    for i in range(_NCHUNK):
        b = i % _NBUF
        gd[i].wait()
        sd[i] = pltpu.async_copy(
            bufs[b], out_hbm.at[pl.ds(base + i * _CHUNK, _CHUNK)], ssems[b]
        )
        nxt = i + _NBUF
        if nxt < _NCHUNK:
            sd[i].wait()
            gd[nxt] = pltpu.async_copy(table_hbm.at[idx_v.at[nxt]], bufs[b], gsems[b])
    for i in range(max(0, _NCHUNK - _NBUF), _NCHUNK):
        sd[i].wait()


def kernel(x, table):
    flat = x.reshape(_TOTAL // _CHUNK, _CHUNK).astype(jnp.int32)
    out = _embed_gather(flat, table)
    return out.reshape(_B, _F, _D)


# speedup vs baseline: 1.6730x; 1.0617x over previous
import functools

import jax
import jax.numpy as jnp
from jax import lax
from jax.experimental import pallas as pl
from jax.experimental.pallas import tpu as pltpu
from jax.experimental.pallas import tpu_sc as plsc

_ROWS = 1000000
_D = 32
_B = 16384
_F = 26
_TOTAL = _B * _F
_NC = 2
_NS = 16
_NW = _NC * _NS
_B_PER_W = _TOTAL // _NW
_CHUNK = 832
_NCHUNK = _B_PER_W // _CHUNK
_NBUF = 4

_mesh = plsc.VectorSubcoreMesh(core_axis_name="c", subcore_axis_name="s")


@functools.partial(
    pl.kernel,
    out_type=jax.ShapeDtypeStruct((_TOTAL, _D), jnp.float32),
    mesh=_mesh,
    scratch_types=[
        pltpu.VMEM((_NCHUNK, _CHUNK), jnp.int32),
        [pltpu.VMEM((_CHUNK, _D), jnp.float32) for _ in range(_NBUF)],
        [pltpu.SemaphoreType.DMA for _ in range(_NBUF)],
        [pltpu.SemaphoreType.DMA for _ in range(_NBUF)],
    ],
    compiler_params=pltpu.CompilerParams(use_tc_tiling_on_sc=False),
)
def _embed_gather(x_hbm, table_hbm, out_hbm, idx_v, bufs, gsems, ssems):
    wid = lax.axis_index("s") * _NC + lax.axis_index("c")
    base = wid * _B_PER_W
    pltpu.sync_copy(x_hbm.at[pl.ds(wid * _NCHUNK, _NCHUNK)], idx_v)

    gd = [None] * _NCHUNK
    sd = [None] * _NCHUNK
    for i in range(min(_NBUF, _NCHUNK)):
        gd[i] = pltpu.async_copy(table_hbm.at[idx_v.at[i]], bufs[i], gsems[i])
    for i in range(_NCHUNK):
        b = i % _NBUF
        gd[i].wait()
        sd[i] = pltpu.async_copy(
            bufs[b], out_hbm.at[pl.ds(base + i * _CHUNK, _CHUNK)], ssems[b]
        )
        nxt = i + _NBUF
        if nxt < _NCHUNK:
            sd[i].wait()
            gd[nxt] = pltpu.async_copy(table_hbm.at[idx_v.at[nxt]], bufs[b], gsems[b])
    for i in range(max(0, _NCHUNK - _NBUF), _NCHUNK):
        sd[i].wait()


def kernel(x, table):
    flat = x.T.reshape(_TOTAL // _CHUNK, _CHUNK).astype(jnp.int32)
    out = _embed_gather(flat, table)
    return out.reshape(_F, _B, _D).transpose(1, 0, 2)
